# baseline (speedup 1.0000x reference)
import functools

import jax
import jax.numpy as jnp
from jax import lax
from jax.experimental import pallas as pl
from jax.experimental.pallas import tpu as pltpu
from jax.experimental.pallas import tpu_sc as plsc

_VOCAB = 100000
_EMBED = 128
_BATCH = 1024
_SEQ = 200
_B = _BATCH * _SEQ
_NC = 2
_NS = 16
_NW = _NC * _NS
_BPW = _B // _NW
_CH = 80
_BLK = 400
_GPB = _BLK // _CH
_NBLK = _BPW // _BLK

_mesh = plsc.VectorSubcoreMesh(
    core_axis_name="c", subcore_axis_name="s", num_cores=_NC, num_subcores=_NS
)


@functools.partial(
    pl.kernel,
    out_type=jax.ShapeDtypeStruct((_B, _EMBED), jnp.float32),
    mesh=_mesh,
    scratch_types=[
        pltpu.VMEM((_BPW,), jnp.int32),
        pltpu.VMEM((2, _BLK, _EMBED), jnp.float32),
        [pltpu.SemaphoreType.DMA] * 2,
        [pltpu.SemaphoreType.DMA] * 2,
    ],
)
def _gather_rows(idx_hbm, table_hbm, out_hbm, idx_v, rows_v, gsems, osems):
    wid = lax.axis_index("s") * _NC + lax.axis_index("c")
    base = wid * _BPW
    pltpu.sync_copy(idx_hbm.at[pl.ds(base, _BPW)], idx_v)

    def gathers_start(k, p):
        descs = []
        for i in range(_GPB):
            off = k * _BLK + i * _CH
            descs.append(
                pltpu.async_copy(
                    table_hbm.at[idx_v.at[pl.ds(off, _CH)]],
                    rows_v.at[p, pl.ds(i * _CH, _CH)],
                    gsems[p],
                )
            )
        return descs

    def out_start(k, p):
        return pltpu.async_copy(
            rows_v.at[p], out_hbm.at[pl.ds(base + k * _BLK, _BLK)], osems[p]
        )

    def out_drain(p):
        pltpu.make_async_copy(
            rows_v.at[p], out_hbm.at[pl.ds(base, _BLK)], osems[p]
        ).wait()

    descs = gathers_start(0, 0)
    for k in range(_NBLK):
        p = k % 2
        if k + 1 < _NBLK:
            if k >= 1:
                out_drain(1 - p)
            next_descs = gathers_start(k + 1, 1 - p)
        else:
            next_descs = None
        for d in descs:
            d.wait()
        out_start(k, p)
        descs = next_descs
    out_drain(0)
    out_drain(1)


def kernel(prompt_ids, embedding_table):
    idx = prompt_ids.reshape(-1).astype(jnp.int32)
    out = _gather_rows(idx, embedding_table)
    return out.reshape(_BATCH, _SEQ, _EMBED)


# speedup vs baseline: 1.0014x; 1.0014x over previous
import functools

import jax
import jax.numpy as jnp
from jax import lax
from jax.experimental import pallas as pl
from jax.experimental.pallas import tpu as pltpu
from jax.experimental.pallas import tpu_sc as plsc

_VOCAB = 100000
_EMBED = 128
_BATCH = 1024
_SEQ = 200
_B = _BATCH * _SEQ
_NC = 2
_NS = 16
_NW = _NC * _NS
_BPW = _B // _NW
_CH = 80
_BLK = 320
_GPB = _BLK // _CH
_NBLK = _BPW // _BLK
_NBUF = 3

_mesh = plsc.VectorSubcoreMesh(
    core_axis_name="c", subcore_axis_name="s", num_cores=_NC, num_subcores=_NS
)


@functools.partial(
    pl.kernel,
    out_type=jax.ShapeDtypeStruct((_B, _EMBED), jnp.float32),
    mesh=_mesh,
    scratch_types=[
        pltpu.VMEM((_BPW,), jnp.int32),
        pltpu.VMEM((_NBUF, _BLK, _EMBED), jnp.float32),
        [pltpu.SemaphoreType.DMA] * _NBUF,
        [pltpu.SemaphoreType.DMA] * _NBUF,
    ],
)
def _gather_rows(idx_hbm, table_hbm, out_hbm, idx_v, rows_v, gsems, osems):
    wid = lax.axis_index("s") * _NC + lax.axis_index("c")
    base = wid * _BPW
    pltpu.sync_copy(idx_hbm.at[pl.ds(base, _BPW)], idx_v)

    def gathers_start(k, p):
        descs = []
        for i in range(_GPB):
            off = k * _BLK + i * _CH
            descs.append(
                pltpu.async_copy(
                    table_hbm.at[idx_v.at[pl.ds(off, _CH)]],
                    rows_v.at[p, pl.ds(i * _CH, _CH)],
                    gsems[p],
                )
            )
        return descs

    def out_start(k, p):
        return pltpu.async_copy(
            rows_v.at[p], out_hbm.at[pl.ds(base + k * _BLK, _BLK)], osems[p]
        )

    def out_drain(p):
        pltpu.make_async_copy(
            rows_v.at[p], out_hbm.at[pl.ds(base, _BLK)], osems[p]
        ).wait()

    descs = {0: gathers_start(0, 0)}
    if _NBLK > 1:
        descs[1] = gathers_start(1, 1)
    for k in range(_NBLK):
        p = k % _NBUF
        nxt = k + 2
        if nxt < _NBLK:
            q = nxt % _NBUF
            if nxt >= _NBUF:
                out_drain(q)
            descs[nxt] = gathers_start(nxt, q)
        for d in descs.pop(k):
            d.wait()
        out_start(k, p)
    for p in range(_NBUF):
        out_drain(p)


def kernel(prompt_ids, embedding_table):
    idx = prompt_ids.reshape(-1).astype(jnp.int32)
    out = _gather_rows(idx, embedding_table)
    return out.reshape(_BATCH, _SEQ, _EMBED)


# speedup vs baseline: 1.0062x; 1.0047x over previous
import functools

import jax
import jax.numpy as jnp
from jax import lax
from jax.experimental import pallas as pl
from jax.experimental.pallas import tpu as pltpu
from jax.experimental.pallas import tpu_sc as plsc

_VOCAB = 100000
_EMBED = 128
_BATCH = 1024
_SEQ = 200
_B = _BATCH * _SEQ
_NC = 2
_NS = 16
_NW = _NC * _NS
_BPW = _B // _NW
_CH = 80
_NCHUNK = _BPW // _CH
_NBUF = 8
_NITER = _NCHUNK // _NBUF

_mesh = plsc.VectorSubcoreMesh(
    core_axis_name="c", subcore_axis_name="s", num_cores=_NC, num_subcores=_NS
)


@functools.partial(
    pl.kernel,
    out_type=jax.ShapeDtypeStruct((_B, _EMBED), jnp.float32),
    mesh=_mesh,
    scratch_types=[
        pltpu.VMEM((_BPW,), jnp.int32),
        pltpu.VMEM((_NBUF, _CH, _EMBED), jnp.float32),
        [pltpu.SemaphoreType.DMA] * _NBUF,
        [pltpu.SemaphoreType.DMA] * _NBUF,
    ],
)
def _gather_rows(idx_hbm, table_hbm, out_hbm, idx_v, rows_v, gsems, osems):
    wid = lax.axis_index("s") * _NC + lax.axis_index("c")
    base = wid * _BPW
    pltpu.sync_copy(idx_hbm.at[pl.ds(base, _BPW)], idx_v)

    def gather_start(g, b):
        return pltpu.async_copy(
            table_hbm.at[idx_v.at[pl.ds(g * _CH, _CH)]], rows_v.at[b], gsems[b]
        )

    def out_start(g, b):
        return pltpu.async_copy(
            rows_v.at[b], out_hbm.at[pl.ds(base + g * _CH, _CH)], osems[b]
        )

    def out_drain(b):
        pltpu.make_async_copy(
            rows_v.at[b], out_hbm.at[pl.ds(base, _CH)], osems[b]
        ).wait()

    def ring(j, first):
        gbase = j * _NBUF
        descs = []
        for b in range(_NBUF):
            if not first:
                out_drain(b)
            descs.append(gather_start(gbase + b, b))
        for b in range(_NBUF):
            descs[b].wait()
            out_start(gbase + b, b)

    ring(0, True)
    lax.fori_loop(1, _NITER, lambda j, c: (ring(j, False), c)[1], 0)
    for b in range(_NBUF):
        out_drain(b)


def kernel(prompt_ids, embedding_table):
    if prompt_ids.dtype != jnp.int32:
        prompt_ids = prompt_ids.astype(jnp.int32)
    out = _gather_rows(prompt_ids.reshape(-1), embedding_table)
    return out.reshape(_BATCH, _SEQ, _EMBED)


# speedup vs baseline: 1.0093x; 1.0031x over previous
import functools

import jax
import jax.numpy as jnp
from jax import lax
from jax.experimental import pallas as pl
from jax.experimental.pallas import tpu as pltpu
from jax.experimental.pallas import tpu_sc as plsc

_VOCAB = 100000
_EMBED = 128
_BATCH = 1024
_SEQ = 200
_B = _BATCH * _SEQ
_NC = 2
_NS = 16
_NW = _NC * _NS
_RPW = _BATCH // _NW
_RPB = 2
_NBLK = _RPW // _RPB
_SPLITS = ((0, 104), (104, 96))

_mesh = plsc.VectorSubcoreMesh(
    core_axis_name="c", subcore_axis_name="s", num_cores=_NC, num_subcores=_NS
)


@functools.partial(
    pl.kernel,
    out_type=jax.ShapeDtypeStruct((_B, _EMBED), jnp.float32),
    mesh=_mesh,
    compiler_params=pltpu.CompilerParams(use_tc_tiling_on_sc=False),
    scratch_types=[
        pltpu.VMEM((_RPW, _SEQ), jnp.int32),
        pltpu.VMEM((2, _RPB * _SEQ, _EMBED), jnp.float32),
        [pltpu.SemaphoreType.DMA] * 2,
        [pltpu.SemaphoreType.DMA] * 2,
    ],
)
def _gather_rows(idx_hbm, table_hbm, out_hbm, idx_v, rows_v, gsems, osems):
    wid = lax.axis_index("s") * _NC + lax.axis_index("c")
    row0 = wid * _RPW
    pltpu.sync_copy(idx_hbm.at[pl.ds(row0, _RPW)], idx_v)

    flat0 = row0 * _SEQ
    blk_rows = _RPB * _SEQ

    def gathers_start(k, p):
        descs = []
        for q in range(_RPB):
            r = k * _RPB + q
            for off, n in _SPLITS:
                descs.append(
                    pltpu.async_copy(
                        table_hbm.at[idx_v.at[r, pl.ds(off, n)]],
                        rows_v.at[p, pl.ds(q * _SEQ + off, n)],
                        gsems[p],
                    )
                )
        return descs

    def out_start(k, p):
        return pltpu.async_copy(
            rows_v.at[p], out_hbm.at[pl.ds(flat0 + k * blk_rows, blk_rows)],
            osems[p],
        )

    def out_drain(p):
        pltpu.make_async_copy(
            rows_v.at[p], out_hbm.at[pl.ds(flat0, blk_rows)], osems[p]
        ).wait()

    descs = gathers_start(0, 0)
    for k in range(_NBLK):
        p = k % 2
        if k + 1 < _NBLK:
            if k >= 1:
                out_drain(1 - p)
            next_descs = gathers_start(k + 1, 1 - p)
        else:
            next_descs = None
        for d in descs:
            d.wait()
        out_start(k, p)
        descs = next_descs
    out_drain(0)
    out_drain(1)


def kernel(prompt_ids, embedding_table):
    if prompt_ids.dtype != jnp.int32:
        prompt_ids = prompt_ids.astype(jnp.int32)
    out = _gather_rows(prompt_ids, embedding_table)
    return out.reshape(_BATCH, _SEQ, _EMBED)
